# baseline (speedup 1.0000x reference)
import jax
import jax.numpy as jnp
from jax import lax
from jax.experimental import pallas as pl
from jax.experimental.pallas import tpu as pltpu
from jax.experimental.pallas import tpu_sc as plsc

NNODES = 10000
NPAD = 10240
BLK = 512
NBLK = NPAD // BLK

L = 16
NSC = 2
NSUB = 16
NW = NSC * NSUB
CW = 64
NCH = 16
EB = 128
NB = 82
EPT = NB * EB
ETOT = NW * EPT
EREAL = 320000 + NNODES
NSLICE = NPAD // NSUB


def _mm_body(x_ref, w_ref, o_ref):
    k = pl.program_id(2)

    @pl.when(k == 0)
    def _():
        o_ref[0] = jnp.zeros_like(o_ref[0])

    o_ref[0] += jnp.dot(x_ref[0], w_ref[0, 0],
                        preferred_element_type=jnp.float32)


def matmul_chunked(xc, wc):
    cin, cout = wc.shape[0], wc.shape[1]
    return pl.pallas_call(
        _mm_body,
        grid=(NBLK, cout, cin),
        in_specs=[
            pl.BlockSpec((1, BLK, 128), lambda i, co, k: (k, i, 0)),
            pl.BlockSpec((1, 1, 128, 128), lambda i, co, k: (k, co, 0, 0)),
        ],
        out_specs=pl.BlockSpec((1, BLK, 128), lambda i, co, k: (co, i, 0)),
        out_shape=jax.ShapeDtypeStruct((cout, NPAD, 128), jnp.float32),
    )(xc, wc)


def _gat_sc_body(hc, av, edges, zf, zf16, out, dpart,
                 srcv, dstv, wv, avs, avd, sidx0, sidx1, rows0, rows1,
                 rows16, acc, acc16, gsem0, gsem1, ssem0, ssem1):
    cid = lax.axis_index("c")
    sid = lax.axis_index("s")
    wid = cid * NSUB + sid

    pltpu.sync_copy(edges.at[0, wid], srcv)
    pltpu.sync_copy(edges.at[1, wid], dstv)
    pltpu.sync_copy(av.at[0], avs)
    pltpu.sync_copy(av.at[1], avd)

    def p1(j, carry):
        for k16 in range(EB // L):
            sl = pl.ds(k16 * L, L)
            sv = srcv[j, sl]
            dv = dstv[j, sl]
            e = plsc.load_gather(avs, [sv]) + plsc.load_gather(avd, [dv])
            e = jnp.where(e >= 0.0, e, 0.2 * e)
            w = jnp.exp(e)
            eid = wid * EPT + j * EB + k16 * L + lax.iota(jnp.int32, L)
            w = jnp.where(eid < EREAL, w, 0.0)
            wv[j, sl] = w
        return carry

    lax.fori_loop(0, NB, p1, 0)

    def zr(r, carry):
        rows16[r, :] = jnp.zeros((L,), jnp.float32)
        return carry

    lax.fori_loop(0, EB, zr, 0)
    pltpu.sync_copy(zf16, acc16.at[pl.ds(sid * NSLICE, NSLICE)])
    plsc.subcore_barrier()

    lanes = lax.iota(jnp.int32, L)
    zcol = jnp.zeros((L,), jnp.int32)

    def p2(j, carry):
        for k16 in range(EB // L):
            w16 = wv[j, pl.ds(k16 * L, L)]
            plsc.store_scatter(rows16, [lanes + k16 * L, zcol], w16)
        pltpu.sync_copy(rows16, acc16.at[dstv.at[j]], add=True)
        return carry

    lax.fori_loop(0, NB, p2, 0)
    plsc.subcore_barrier()
    pltpu.sync_copy(acc16.at[pl.ds(sid * NSLICE, NSLICE)],
                    dpart.at[cid, pl.ds(sid * NSLICE, NSLICE)])

    bufs = ((rows0, sidx0, gsem0, ssem0), (rows1, sidx1, gsem1, ssem1))

    def _build_sidx(sb, j, coff):
        for k16 in range(EB // L):
            sl = pl.ds(k16 * L, L)
            sb[sl] = srcv[j, sl] * 2 + coff

    def chunk(c, carry):
        pltpu.sync_copy(zf, acc.at[pl.ds(sid * NSLICE, NSLICE)])
        plsc.subcore_barrier()
        coff = (c >> 1) * (2 * NPAD) + (c & 1)

        for b in range(2):
            rb, sb, gs, _ = bufs[b]
            _build_sidx(sb, b, coff)
            pltpu.async_copy(hc.at[sb], rb, gs)

        def pjj(jj, inner):
            for b in range(2):
                rb, sb, gs, ss = bufs[b]
                j = 2 * jj + b
                pltpu.make_async_copy(hc.at[sb], rb, gs).wait()
                for k16 in range(EB // L):
                    w16 = wv[j, pl.ds(k16 * L, L)]
                    for i in range(L):
                        ws = jnp.full((L,), w16[i], jnp.float32)
                        k = k16 * L + i
                        for q in range(CW // L):
                            qs = pl.ds(q * L, L)
                            rb[k, qs] = rb[k, qs] * ws
                pltpu.async_copy(rb, acc.at[dstv.at[j]], ss, add=True)

                @pl.when(j + 2 < NB)
                def _():
                    pltpu.make_async_copy(rb, acc.at[dstv.at[j]], ss).wait()
                    _build_sidx(sb, j + 2, coff)
                    pltpu.async_copy(hc.at[sb], rb, gs)

            return inner

        lax.fori_loop(0, NB // 2, pjj, 0)
        for b in range(2):
            rb, sb, gs, ss = bufs[b]
            pltpu.make_async_copy(rb, acc.at[dstv.at[NB - 2 + b]], ss).wait()
        plsc.subcore_barrier()
        pltpu.sync_copy(acc.at[pl.ds(sid * NSLICE, NSLICE)],
                        out.at[cid, c, pl.ds(sid * NSLICE, NSLICE)])
        plsc.subcore_barrier()
        return carry

    lax.fori_loop(0, NCH, chunk, 0)


_gat_sc = pl.kernel(
    _gat_sc_body,
    out_type=(jax.ShapeDtypeStruct((NSC, NCH, NPAD, CW), jnp.float32),
              jax.ShapeDtypeStruct((NSC, NPAD, 16), jnp.float32)),
    mesh=plsc.VectorSubcoreMesh(core_axis_name="c", subcore_axis_name="s"),
    compiler_params=pltpu.CompilerParams(needs_layout_passes=False,
                                         use_tc_tiling_on_sc=False),
    scratch_types=[
        pltpu.VMEM((NB, EB), jnp.int32),
        pltpu.VMEM((NB, EB), jnp.int32),
        pltpu.VMEM((NB, EB), jnp.float32),
        pltpu.VMEM((NPAD,), jnp.float32),
        pltpu.VMEM((NPAD,), jnp.float32),
        pltpu.VMEM((EB,), jnp.int32),
        pltpu.VMEM((EB,), jnp.int32),
        pltpu.VMEM((EB, CW), jnp.float32),
        pltpu.VMEM((EB, CW), jnp.float32),
        pltpu.VMEM((EB, 16), jnp.float32),
        pltpu.VMEM_SHARED((NPAD, CW), jnp.float32),
        pltpu.VMEM_SHARED((NPAD, 16), jnp.float32),
        pltpu.SemaphoreType.DMA,
        pltpu.SemaphoreType.DMA,
        pltpu.SemaphoreType.DMA,
        pltpu.SemaphoreType.DMA,
    ],
)


def _combine_body(osc_ref, dp_ref, b_ref, h_ref, s_ref, q_ref):
    i = pl.program_id(0)
    o = osc_ref[...]
    s64 = o[0] + o[1]
    s = jnp.stack([jnp.concatenate([s64[2 * c], s64[2 * c + 1]], axis=-1)
                   for c in range(8)])
    dp = dp_ref[...]
    den = (dp[0, :, 0:1] + dp[1, :, 0:1])[None]
    h = s / den + b_ref[...]
    row = i * BLK + jax.lax.broadcasted_iota(jnp.int32, s.shape, 1)
    h = jnp.where(row < NNODES, h, 0.0)
    h_ref[...] = h

    @pl.when(i == 0)
    def _():
        s_ref[...] = jnp.zeros_like(s_ref)
        q_ref[...] = jnp.zeros_like(q_ref)

    s_ref[...] += jnp.sum(h, axis=1, keepdims=True)
    q_ref[...] += jnp.sum(h * h, axis=1, keepdims=True)


def combine_bias_stats(osc, dpart, b):
    return pl.pallas_call(
        _combine_body,
        grid=(NBLK,),
        in_specs=[
            pl.BlockSpec((NSC, NCH, BLK, CW), lambda i: (0, 0, i, 0)),
            pl.BlockSpec((NSC, BLK, 16), lambda i: (0, i, 0)),
            pl.BlockSpec((8, 1, 128), lambda i: (0, 0, 0)),
        ],
        out_specs=[
            pl.BlockSpec((8, BLK, 128), lambda i: (0, i, 0)),
            pl.BlockSpec((8, 1, 128), lambda i: (0, 0, 0)),
            pl.BlockSpec((8, 1, 128), lambda i: (0, 0, 0)),
        ],
        out_shape=[
            jax.ShapeDtypeStruct((8, NPAD, 128), jnp.float32),
            jax.ShapeDtypeStruct((8, 1, 128), jnp.float32),
            jax.ShapeDtypeStruct((8, 1, 128), jnp.float32),
        ],
    )(osc, dpart, b)


def _bn_apply_body(h_ref, m_ref, r_ref, g_ref, b_ref, o_ref):
    x = (h_ref[...] - m_ref[...]) * r_ref[...]
    o_ref[...] = jnp.maximum(x * g_ref[...] + b_ref[...], 0.0)


def bn_apply(h, mean, rstd, g, beta):
    vec = pl.BlockSpec((8, 1, 128), lambda i: (0, 0, 0))
    return pl.pallas_call(
        _bn_apply_body,
        grid=(NBLK,),
        in_specs=[pl.BlockSpec((8, BLK, 128), lambda i: (0, i, 0)),
                  vec, vec, vec, vec],
        out_specs=pl.BlockSpec((8, BLK, 128), lambda i: (0, i, 0)),
        out_shape=jax.ShapeDtypeStruct((8, NPAD, 128), jnp.float32),
    )(h, mean, rstd, g, beta)


def _head_body(h_ref, w1_ref, b1_ref, w2_ref, b2_ref, o_ref, a_ref):
    k = pl.program_id(1)

    @pl.when(k == 0)
    def _():
        a_ref[...] = jnp.zeros_like(a_ref)

    a_ref[...] += jnp.dot(h_ref[0], w1_ref[0],
                          preferred_element_type=jnp.float32)

    @pl.when(k == 7)
    def _():
        a = jnp.maximum(a_ref[...] + b1_ref[...], 0.0)
        z = jnp.dot(a, w2_ref[...], preferred_element_type=jnp.float32)
        z = z + b2_ref[...]
        col = jax.lax.broadcasted_iota(jnp.int32, z.shape, 1)
        valid = col < 7
        zm = jnp.where(valid, z, -jnp.inf)
        m = jnp.max(zm, axis=1, keepdims=True)
        ssum = jnp.sum(jnp.where(valid, jnp.exp(z - m), 0.0),
                       axis=1, keepdims=True)
        o_ref[...] = z - m - jnp.log(ssum)


def head(h, lw1, lb1, lw2, lb2):
    dmid = lw1.shape[1]
    lw1c = lw1.reshape(8, 128, dmid)
    lw2p = jnp.zeros((dmid, 128), jnp.float32).at[:, :7].set(lw2)
    lb2p = jnp.zeros((1, 128), jnp.float32).at[0, :7].set(lb2)
    out = pl.pallas_call(
        _head_body,
        grid=(NBLK, 8),
        in_specs=[
            pl.BlockSpec((1, BLK, 128), lambda i, k: (k, i, 0)),
            pl.BlockSpec((1, 128, dmid), lambda i, k: (k, 0, 0)),
            pl.BlockSpec((1, dmid), lambda i, k: (0, 0)),
            pl.BlockSpec((dmid, 128), lambda i, k: (0, 0)),
            pl.BlockSpec((1, 128), lambda i, k: (0, 0)),
        ],
        out_specs=pl.BlockSpec((BLK, 128), lambda i, k: (i, 0)),
        out_shape=jax.ShapeDtypeStruct((NPAD, 128), jnp.float32),
        scratch_shapes=[pltpu.VMEM((BLK, dmid), jnp.float32)],
    )(h, lw1c, lb1.reshape(1, -1), lw2p, lb2p)
    return out[:NNODES, :7]


def _weight_chunks(W, a_src, a_dst):
    d = W.shape[0]
    aux = jnp.zeros((d, 128), jnp.float32)
    aux = aux.at[:, 0].set(W @ a_src).at[:, 1].set(W @ a_dst)
    w_aug = jnp.concatenate([W, aux], axis=1)
    return w_aug.reshape(d // 128, 128, 9, 128).transpose(0, 2, 1, 3)


def gat_layer(xc, edges, zf, zf16, W, a_src, a_dst, b):
    wc = _weight_chunks(W, a_src, a_dst)
    hc = matmul_chunked(xc, wc)
    av = jnp.stack([hc[8, :, 0], hc[8, :, 1]])
    hcflat = hc.reshape(9 * NPAD * 2, CW)
    osc, dpart = _gat_sc(hcflat, av, edges, zf, zf16)
    return combine_bias_stats(osc, dpart, b.reshape(8, 1, 128))


def _bn_stats(s, q):
    mean = s / NNODES
    var = q / NNODES - mean * mean
    return mean, jax.lax.rsqrt(var + 1e-5)


def kernel(x, edge_index, W1, a_src1, a_dst1, b1, g1, beta1,
           W2, a_src2, a_dst2, b2, g2, beta2, lw1, lb1, lw2, lb2):
    loop = jnp.arange(NNODES, dtype=edge_index.dtype)
    src = jnp.concatenate([edge_index[0], loop])
    dst = jnp.concatenate([edge_index[1], loop])
    pad = ETOT - src.shape[0]
    src = jnp.pad(src, (0, pad))
    dst = jnp.pad(dst, (0, pad))
    edges = jnp.stack([src, dst]).reshape(2, NW, NB, EB)
    zf = jnp.zeros((NSLICE, CW), jnp.float32)
    zf16 = jnp.zeros((NSLICE, 16), jnp.float32)

    xc = jnp.pad(x, ((0, NPAD - NNODES), (0, 0)))[None]
    h, s, q = gat_layer(xc, edges, zf, zf16, W1, a_src1, a_dst1, b1)
    mean, rstd = _bn_stats(s, q)
    h = bn_apply(h, mean, rstd, g1.reshape(8, 1, 128), beta1.reshape(8, 1, 128))
    h, s, q = gat_layer(h, edges, zf, zf16, W2, a_src2, a_dst2, b2)
    mean, rstd = _bn_stats(s, q)
    h = bn_apply(h, mean, rstd, g2.reshape(8, 1, 128), beta2.reshape(8, 1, 128))
    return head(h, lw1, lb1, lw2, lb2)


# speedup vs baseline: 1.4196x; 1.4196x over previous
import jax
import jax.numpy as jnp
from jax import lax
from jax.experimental import pallas as pl
from jax.experimental.pallas import tpu as pltpu
from jax.experimental.pallas import tpu_sc as plsc

NNODES = 10000
NPAD = 10240
BLK = 512
NBLK = NPAD // BLK

L = 16
NSC = 2
NSUB = 16
NW = NSC * NSUB
CW = 64
NCH = 16
EB = 192
NB = 54
EPT = NB * EB
ETOT = NW * EPT
EREAL = 320000 + NNODES
NSLICE = NPAD // NSUB


def _mm_body(x_ref, w_ref, o_ref):
    k = pl.program_id(2)

    @pl.when(k == 0)
    def _():
        o_ref[0] = jnp.zeros_like(o_ref[0])

    o_ref[0] += jnp.dot(x_ref[0], w_ref[0, 0],
                        preferred_element_type=jnp.float32)


def matmul_chunked(xc, wc):
    cin, cout = wc.shape[0], wc.shape[1]
    return pl.pallas_call(
        _mm_body,
        grid=(NBLK, cout, cin),
        in_specs=[
            pl.BlockSpec((1, BLK, 128), lambda i, co, k: (k, i, 0)),
            pl.BlockSpec((1, 1, 128, 128), lambda i, co, k: (k, co, 0, 0)),
        ],
        out_specs=pl.BlockSpec((1, BLK, 128), lambda i, co, k: (co, i, 0)),
        out_shape=jax.ShapeDtypeStruct((cout, NPAD, 128), jnp.float32),
    )(xc, wc)


def _gat_sc_body(hc, av, edges, zf, zf16, out, dpart,
                 srcv, dstv, wv, avs, avd, sidx0, sidx1, rows0, rows1,
                 rows16, acc, acc16, gsem0, gsem1):
    cid = lax.axis_index("c")
    sid = lax.axis_index("s")
    wid = cid * NSUB + sid

    pltpu.sync_copy(edges.at[0, wid], srcv)
    pltpu.sync_copy(edges.at[1, wid], dstv)
    pltpu.sync_copy(av.at[0], avs)
    pltpu.sync_copy(av.at[1], avd)

    def p1(j, carry):
        for k16 in range(EB // L):
            sl = pl.ds(k16 * L, L)
            sv = srcv[j, sl]
            dv = dstv[j, sl]
            e = plsc.load_gather(avs, [sv]) + plsc.load_gather(avd, [dv])
            e = jnp.where(e >= 0.0, e, 0.2 * e)
            w = jnp.exp(e)
            eid = wid * EPT + j * EB + k16 * L + lax.iota(jnp.int32, L)
            w = jnp.where(eid < EREAL, w, 0.0)
            wv[j, sl] = w
        return carry

    lax.fori_loop(0, NB, p1, 0)

    def zr(r, carry):
        rows16[r, :] = jnp.zeros((L,), jnp.float32)
        return carry

    lax.fori_loop(0, EB, zr, 0)
    pltpu.sync_copy(zf16, acc16.at[pl.ds(sid * NSLICE, NSLICE)])
    plsc.subcore_barrier()

    lanes = lax.iota(jnp.int32, L)
    zcol = jnp.zeros((L,), jnp.int32)

    def p2(j, carry):
        for k16 in range(EB // L):
            w16 = wv[j, pl.ds(k16 * L, L)]
            plsc.store_scatter(rows16, [lanes + k16 * L, zcol], w16)
        pltpu.sync_copy(rows16, acc16.at[dstv.at[j]], add=True)
        return carry

    lax.fori_loop(0, NB, p2, 0)
    plsc.subcore_barrier()
    pltpu.sync_copy(acc16.at[pl.ds(sid * NSLICE, NSLICE)],
                    dpart.at[cid, pl.ds(sid * NSLICE, NSLICE)])

    bufs = ((rows0, sidx0, gsem0), (rows1, sidx1, gsem1))

    def _build_sidx(sb, j, coff):
        for k16 in range(EB // L):
            sl = pl.ds(k16 * L, L)
            sb[sl] = srcv[j, sl] * 2 + coff

    def chunk(c, carry):
        pltpu.sync_copy(zf, acc.at[pl.ds(sid * NSLICE, NSLICE)])
        plsc.subcore_barrier()
        coff = (c >> 1) * (2 * NPAD) + (c & 1)

        for b in range(2):
            rb, sb, gs = bufs[b]
            _build_sidx(sb, b, coff)
            pltpu.async_copy(hc.at[sb], rb, gs)

        def pjj(jj, inner):
            for b in range(2):
                rb, sb, gs = bufs[b]
                j = 2 * jj + b
                pltpu.make_async_copy(hc.at[sb], rb, gs).wait()
                for k16 in range(EB // L):
                    w16 = wv[j, pl.ds(k16 * L, L)]
                    for i in range(L):
                        ws = jnp.full((L,), w16[i], jnp.float32)
                        k = k16 * L + i
                        for q in range(CW // L):
                            qs = pl.ds(q * L, L)
                            rb[k, qs] = rb[k, qs] * ws
                pltpu.sync_copy(rb, acc.at[dstv.at[j]], add=True)

                @pl.when(j + 2 < NB)
                def _():
                    _build_sidx(sb, j + 2, coff)
                    pltpu.async_copy(hc.at[sb], rb, gs)

            return inner

        lax.fori_loop(0, NB // 2, pjj, 0)
        plsc.subcore_barrier()
        pltpu.sync_copy(acc.at[pl.ds(sid * NSLICE, NSLICE)],
                        out.at[cid, c, pl.ds(sid * NSLICE, NSLICE)])
        plsc.subcore_barrier()
        return carry

    lax.fori_loop(0, NCH, chunk, 0)


_gat_sc = pl.kernel(
    _gat_sc_body,
    out_type=(jax.ShapeDtypeStruct((NSC, NCH, NPAD, CW), jnp.float32),
              jax.ShapeDtypeStruct((NSC, NPAD, 16), jnp.float32)),
    mesh=plsc.VectorSubcoreMesh(core_axis_name="c", subcore_axis_name="s"),
    compiler_params=pltpu.CompilerParams(needs_layout_passes=False,
                                         use_tc_tiling_on_sc=False),
    scratch_types=[
        pltpu.VMEM((NB, EB), jnp.int32),
        pltpu.VMEM((NB, EB), jnp.int32),
        pltpu.VMEM((NB, EB), jnp.float32),
        pltpu.VMEM((NPAD,), jnp.float32),
        pltpu.VMEM((NPAD,), jnp.float32),
        pltpu.VMEM((EB,), jnp.int32),
        pltpu.VMEM((EB,), jnp.int32),
        pltpu.VMEM((EB, CW), jnp.float32),
        pltpu.VMEM((EB, CW), jnp.float32),
        pltpu.VMEM((EB, 16), jnp.float32),
        pltpu.VMEM_SHARED((NPAD, CW), jnp.float32),
        pltpu.VMEM_SHARED((NPAD, 16), jnp.float32),
        pltpu.SemaphoreType.DMA,
        pltpu.SemaphoreType.DMA,
    ],
)


def _combine_body(osc_ref, dp_ref, b_ref, h_ref, s_ref, q_ref):
    i = pl.program_id(0)
    o = osc_ref[...]
    s64 = o[0] + o[1]
    s = jnp.stack([jnp.concatenate([s64[2 * c], s64[2 * c + 1]], axis=-1)
                   for c in range(8)])
    dp = dp_ref[...]
    den = (dp[0, :, 0:1] + dp[1, :, 0:1])[None]
    h = s / den + b_ref[...]
    row = i * BLK + jax.lax.broadcasted_iota(jnp.int32, s.shape, 1)
    h = jnp.where(row < NNODES, h, 0.0)
    h_ref[...] = h

    @pl.when(i == 0)
    def _():
        s_ref[...] = jnp.zeros_like(s_ref)
        q_ref[...] = jnp.zeros_like(q_ref)

    s_ref[...] += jnp.sum(h, axis=1, keepdims=True)
    q_ref[...] += jnp.sum(h * h, axis=1, keepdims=True)


def combine_bias_stats(osc, dpart, b):
    return pl.pallas_call(
        _combine_body,
        grid=(NBLK,),
        in_specs=[
            pl.BlockSpec((NSC, NCH, BLK, CW), lambda i: (0, 0, i, 0)),
            pl.BlockSpec((NSC, BLK, 16), lambda i: (0, i, 0)),
            pl.BlockSpec((8, 1, 128), lambda i: (0, 0, 0)),
        ],
        out_specs=[
            pl.BlockSpec((8, BLK, 128), lambda i: (0, i, 0)),
            pl.BlockSpec((8, 1, 128), lambda i: (0, 0, 0)),
            pl.BlockSpec((8, 1, 128), lambda i: (0, 0, 0)),
        ],
        out_shape=[
            jax.ShapeDtypeStruct((8, NPAD, 128), jnp.float32),
            jax.ShapeDtypeStruct((8, 1, 128), jnp.float32),
            jax.ShapeDtypeStruct((8, 1, 128), jnp.float32),
        ],
    )(osc, dpart, b)


def _bn_apply_body(h_ref, m_ref, r_ref, g_ref, b_ref, o_ref):
    x = (h_ref[...] - m_ref[...]) * r_ref[...]
    o_ref[...] = jnp.maximum(x * g_ref[...] + b_ref[...], 0.0)


def bn_apply(h, mean, rstd, g, beta):
    vec = pl.BlockSpec((8, 1, 128), lambda i: (0, 0, 0))
    return pl.pallas_call(
        _bn_apply_body,
        grid=(NBLK,),
        in_specs=[pl.BlockSpec((8, BLK, 128), lambda i: (0, i, 0)),
                  vec, vec, vec, vec],
        out_specs=pl.BlockSpec((8, BLK, 128), lambda i: (0, i, 0)),
        out_shape=jax.ShapeDtypeStruct((8, NPAD, 128), jnp.float32),
    )(h, mean, rstd, g, beta)


def _head_body(h_ref, w1_ref, b1_ref, w2_ref, b2_ref, o_ref, a_ref):
    k = pl.program_id(1)

    @pl.when(k == 0)
    def _():
        a_ref[...] = jnp.zeros_like(a_ref)

    a_ref[...] += jnp.dot(h_ref[0], w1_ref[0],
                          preferred_element_type=jnp.float32)

    @pl.when(k == 7)
    def _():
        a = jnp.maximum(a_ref[...] + b1_ref[...], 0.0)
        z = jnp.dot(a, w2_ref[...], preferred_element_type=jnp.float32)
        z = z + b2_ref[...]
        col = jax.lax.broadcasted_iota(jnp.int32, z.shape, 1)
        valid = col < 7
        zm = jnp.where(valid, z, -jnp.inf)
        m = jnp.max(zm, axis=1, keepdims=True)
        ssum = jnp.sum(jnp.where(valid, jnp.exp(z - m), 0.0),
                       axis=1, keepdims=True)
        o_ref[...] = z - m - jnp.log(ssum)


def head(h, lw1, lb1, lw2, lb2):
    dmid = lw1.shape[1]
    lw1c = lw1.reshape(8, 128, dmid)
    lw2p = jnp.zeros((dmid, 128), jnp.float32).at[:, :7].set(lw2)
    lb2p = jnp.zeros((1, 128), jnp.float32).at[0, :7].set(lb2)
    out = pl.pallas_call(
        _head_body,
        grid=(NBLK, 8),
        in_specs=[
            pl.BlockSpec((1, BLK, 128), lambda i, k: (k, i, 0)),
            pl.BlockSpec((1, 128, dmid), lambda i, k: (k, 0, 0)),
            pl.BlockSpec((1, dmid), lambda i, k: (0, 0)),
            pl.BlockSpec((dmid, 128), lambda i, k: (0, 0)),
            pl.BlockSpec((1, 128), lambda i, k: (0, 0)),
        ],
        out_specs=pl.BlockSpec((BLK, 128), lambda i, k: (i, 0)),
        out_shape=jax.ShapeDtypeStruct((NPAD, 128), jnp.float32),
        scratch_shapes=[pltpu.VMEM((BLK, dmid), jnp.float32)],
    )(h, lw1c, lb1.reshape(1, -1), lw2p, lb2p)
    return out[:NNODES, :7]


def _weight_chunks(W, a_src, a_dst):
    d = W.shape[0]
    aux = jnp.zeros((d, 128), jnp.float32)
    aux = aux.at[:, 0].set(W @ a_src).at[:, 1].set(W @ a_dst)
    w_aug = jnp.concatenate([W, aux], axis=1)
    return w_aug.reshape(d // 128, 128, 9, 128).transpose(0, 2, 1, 3)


def gat_layer(xc, edges, zf, zf16, W, a_src, a_dst, b):
    wc = _weight_chunks(W, a_src, a_dst)
    hc = matmul_chunked(xc, wc)
    av = jnp.stack([hc[8, :, 0], hc[8, :, 1]])
    hcflat = hc.reshape(9 * NPAD * 2, CW)
    osc, dpart = _gat_sc(hcflat, av, edges, zf, zf16)
    return combine_bias_stats(osc, dpart, b.reshape(8, 1, 128))


def _bn_stats(s, q):
    mean = s / NNODES
    var = q / NNODES - mean * mean
    return mean, jax.lax.rsqrt(var + 1e-5)


def kernel(x, edge_index, W1, a_src1, a_dst1, b1, g1, beta1,
           W2, a_src2, a_dst2, b2, g2, beta2, lw1, lb1, lw2, lb2):
    loop = jnp.arange(NNODES, dtype=edge_index.dtype)
    src = jnp.concatenate([edge_index[0], loop])
    dst = jnp.concatenate([edge_index[1], loop])
    pad = ETOT - src.shape[0]
    src = jnp.pad(src, (0, pad))
    dst = jnp.pad(dst, (0, pad))
    edges = jnp.stack([src, dst]).reshape(2, NW, NB, EB)
    zf = jnp.zeros((NSLICE, CW), jnp.float32)
    zf16 = jnp.zeros((NSLICE, 16), jnp.float32)

    xc = jnp.pad(x, ((0, NPAD - NNODES), (0, 0)))[None]
    h, s, q = gat_layer(xc, edges, zf, zf16, W1, a_src1, a_dst1, b1)
    mean, rstd = _bn_stats(s, q)
    h = bn_apply(h, mean, rstd, g1.reshape(8, 1, 128), beta1.reshape(8, 1, 128))
    h, s, q = gat_layer(h, edges, zf, zf16, W2, a_src2, a_dst2, b2)
    mean, rstd = _bn_stats(s, q)
    h = bn_apply(h, mean, rstd, g2.reshape(8, 1, 128), beta2.reshape(8, 1, 128))
    return head(h, lw1, lb1, lw2, lb2)


# speedup vs baseline: 1.7390x; 1.2250x over previous
import jax
import jax.numpy as jnp
from jax import lax
from jax.experimental import pallas as pl
from jax.experimental.pallas import tpu as pltpu
from jax.experimental.pallas import tpu_sc as plsc

NNODES = 10000
NPAD = 10240
BLK = 512
NBLK = NPAD // BLK

L = 16
NSC = 2
NSUB = 16
NW = NSC * NSUB
CW = 64
NCH = 16
EB = 192
NB = 54
EPT = NB * EB
ETOT = NW * EPT
EREAL = 320000 + NNODES
NSLICE = NPAD // NSUB


def matmul_chunked(xc, wc):
    cin, cout = wc.shape[0], wc.shape[1]

    def body(x_ref, w_ref, o_ref):
        for co in range(cout):
            acc = jnp.zeros((BLK, 128), jnp.float32)
            for k in range(cin):
                acc += jnp.dot(x_ref[k], w_ref[k, co],
                               preferred_element_type=jnp.float32)
            o_ref[co] = acc

    return pl.pallas_call(
        body,
        grid=(NBLK,),
        in_specs=[
            pl.BlockSpec((cin, BLK, 128), lambda i: (0, i, 0)),
            pl.BlockSpec((cin, cout, 128, 128), lambda i: (0, 0, 0, 0)),
        ],
        out_specs=pl.BlockSpec((cout, BLK, 128), lambda i: (0, i, 0)),
        out_shape=jax.ShapeDtypeStruct((cout, NPAD, 128), jnp.float32),
    )(xc, wc)


def _gat_sc_body(hc, av, edges, zf, zf16, out, dpart,
                 srcv, dstv, wv, avs, avd, sidx0, sidx1, rows0, rows1,
                 rows16, acc, acc16, gsem0, gsem1):
    cid = lax.axis_index("c")
    sid = lax.axis_index("s")
    wid = cid * NSUB + sid

    pltpu.sync_copy(edges.at[0, wid], srcv)
    pltpu.sync_copy(edges.at[1, wid], dstv)
    pltpu.sync_copy(av.at[0], avs)
    pltpu.sync_copy(av.at[1], avd)

    def p1(j, carry):
        for k16 in range(EB // L):
            sl = pl.ds(k16 * L, L)
            sv = srcv[j, sl]
            dv = dstv[j, sl]
            e = plsc.load_gather(avs, [sv]) + plsc.load_gather(avd, [dv])
            e = jnp.where(e >= 0.0, e, 0.2 * e)
            w = jnp.exp(e)
            eid = wid * EPT + j * EB + k16 * L + lax.iota(jnp.int32, L)
            w = jnp.where(eid < EREAL, w, 0.0)
            wv[j, sl] = w
        return carry

    lax.fori_loop(0, NB, p1, 0)

    def zr(r, carry):
        rows16[r, :] = jnp.zeros((L,), jnp.float32)
        return carry

    lax.fori_loop(0, EB, zr, 0)
    pltpu.sync_copy(zf16, acc16.at[pl.ds(sid * NSLICE, NSLICE)])
    plsc.subcore_barrier()

    lanes = lax.iota(jnp.int32, L)
    zcol = jnp.zeros((L,), jnp.int32)

    def p2(j, carry):
        for k16 in range(EB // L):
            w16 = wv[j, pl.ds(k16 * L, L)]
            plsc.store_scatter(rows16, [lanes + k16 * L, zcol], w16)
        pltpu.sync_copy(rows16, acc16.at[dstv.at[j]], add=True)
        return carry

    lax.fori_loop(0, NB, p2, 0)
    plsc.subcore_barrier()
    pltpu.sync_copy(acc16.at[pl.ds(sid * NSLICE, NSLICE)],
                    dpart.at[cid, pl.ds(sid * NSLICE, NSLICE)])

    bufs = ((rows0, sidx0, gsem0), (rows1, sidx1, gsem1))

    def _build_sidx(sb, j, coff):
        for k16 in range(EB // L):
            sl = pl.ds(k16 * L, L)
            sb[sl] = srcv[j, sl] * 2 + coff

    def chunk(c, carry):
        pltpu.sync_copy(zf, acc.at[pl.ds(sid * NSLICE, NSLICE)])
        plsc.subcore_barrier()
        coff = (c >> 1) * (2 * NPAD) + (c & 1)

        for b in range(2):
            rb, sb, gs = bufs[b]
            _build_sidx(sb, b, coff)
            pltpu.async_copy(hc.at[sb], rb, gs)

        def pjj(jj, inner):
            for b in range(2):
                rb, sb, gs = bufs[b]
                j = 2 * jj + b
                pltpu.make_async_copy(hc.at[sb], rb, gs).wait()
                for k16 in range(EB // L):
                    w16 = wv[j, pl.ds(k16 * L, L)]
                    for i in range(L):
                        ws = jnp.full((L,), w16[i], jnp.float32)
                        k = k16 * L + i
                        for q in range(CW // L):
                            qs = pl.ds(q * L, L)
                            rb[k, qs] = rb[k, qs] * ws
                pltpu.sync_copy(rb, acc.at[dstv.at[j]], add=True)

                @pl.when(j + 2 < NB)
                def _():
                    _build_sidx(sb, j + 2, coff)
                    pltpu.async_copy(hc.at[sb], rb, gs)

            return inner

        lax.fori_loop(0, NB // 2, pjj, 0)
        plsc.subcore_barrier()
        pltpu.sync_copy(acc.at[pl.ds(sid * NSLICE, NSLICE)],
                        out.at[cid, c, pl.ds(sid * NSLICE, NSLICE)])
        plsc.subcore_barrier()
        return carry

    lax.fori_loop(0, NCH, chunk, 0)


_gat_sc = pl.kernel(
    _gat_sc_body,
    out_type=(jax.ShapeDtypeStruct((NSC, NCH, NPAD, CW), jnp.float32),
              jax.ShapeDtypeStruct((NSC, NPAD, 16), jnp.float32)),
    mesh=plsc.VectorSubcoreMesh(core_axis_name="c", subcore_axis_name="s"),
    compiler_params=pltpu.CompilerParams(needs_layout_passes=False,
                                         use_tc_tiling_on_sc=False),
    scratch_types=[
        pltpu.VMEM((NB, EB), jnp.int32),
        pltpu.VMEM((NB, EB), jnp.int32),
        pltpu.VMEM((NB, EB), jnp.float32),
        pltpu.VMEM((NPAD,), jnp.float32),
        pltpu.VMEM((NPAD,), jnp.float32),
        pltpu.VMEM((EB,), jnp.int32),
        pltpu.VMEM((EB,), jnp.int32),
        pltpu.VMEM((EB, CW), jnp.float32),
        pltpu.VMEM((EB, CW), jnp.float32),
        pltpu.VMEM((EB, 16), jnp.float32),
        pltpu.VMEM_SHARED((NPAD, CW), jnp.float32),
        pltpu.VMEM_SHARED((NPAD, 16), jnp.float32),
        pltpu.SemaphoreType.DMA,
        pltpu.SemaphoreType.DMA,
    ],
)


def _combine_body(osc_ref, dp_ref, b_ref, h_ref, s_ref, q_ref):
    i = pl.program_id(0)
    o = osc_ref[...]
    s64 = o[0] + o[1]
    s = jnp.stack([jnp.concatenate([s64[2 * c], s64[2 * c + 1]], axis=-1)
                   for c in range(8)])
    dp = dp_ref[...]
    den = (dp[0, :, 0:1] + dp[1, :, 0:1])[None]
    h = s / den + b_ref[...]
    row = i * BLK + jax.lax.broadcasted_iota(jnp.int32, s.shape, 1)
    h = jnp.where(row < NNODES, h, 0.0)
    h_ref[...] = h

    @pl.when(i == 0)
    def _():
        s_ref[...] = jnp.zeros_like(s_ref)
        q_ref[...] = jnp.zeros_like(q_ref)

    s_ref[...] += jnp.sum(h, axis=1, keepdims=True)
    q_ref[...] += jnp.sum(h * h, axis=1, keepdims=True)


def combine_bias_stats(osc, dpart, b):
    return pl.pallas_call(
        _combine_body,
        grid=(NBLK,),
        in_specs=[
            pl.BlockSpec((NSC, NCH, BLK, CW), lambda i: (0, 0, i, 0)),
            pl.BlockSpec((NSC, BLK, 16), lambda i: (0, i, 0)),
            pl.BlockSpec((8, 1, 128), lambda i: (0, 0, 0)),
        ],
        out_specs=[
            pl.BlockSpec((8, BLK, 128), lambda i: (0, i, 0)),
            pl.BlockSpec((8, 1, 128), lambda i: (0, 0, 0)),
            pl.BlockSpec((8, 1, 128), lambda i: (0, 0, 0)),
        ],
        out_shape=[
            jax.ShapeDtypeStruct((8, NPAD, 128), jnp.float32),
            jax.ShapeDtypeStruct((8, 1, 128), jnp.float32),
            jax.ShapeDtypeStruct((8, 1, 128), jnp.float32),
        ],
    )(osc, dpart, b)


def _bn_apply_body(h_ref, m_ref, r_ref, g_ref, b_ref, o_ref):
    x = (h_ref[...] - m_ref[...]) * r_ref[...]
    o_ref[...] = jnp.maximum(x * g_ref[...] + b_ref[...],
                             0.0).astype(jnp.bfloat16)


def bn_apply(h, mean, rstd, g, beta):
    vec = pl.BlockSpec((8, 1, 128), lambda i: (0, 0, 0))
    return pl.pallas_call(
        _bn_apply_body,
        grid=(NBLK,),
        in_specs=[pl.BlockSpec((8, BLK, 128), lambda i: (0, i, 0)),
                  vec, vec, vec, vec],
        out_specs=pl.BlockSpec((8, BLK, 128), lambda i: (0, i, 0)),
        out_shape=jax.ShapeDtypeStruct((8, NPAD, 128), jnp.bfloat16),
    )(h, mean, rstd, g, beta)


def _head_body(h_ref, w1_ref, b1_ref, w2_ref, b2_ref, o_ref):
    a = jnp.zeros((BLK, 256), jnp.float32)
    for k in range(8):
        a += jnp.dot(h_ref[k], w1_ref[k], preferred_element_type=jnp.float32)
    a = jnp.maximum(a + b1_ref[...], 0.0).astype(jnp.bfloat16)
    z = jnp.dot(a, w2_ref[...], preferred_element_type=jnp.float32)
    z = z + b2_ref[...]
    col = jax.lax.broadcasted_iota(jnp.int32, z.shape, 1)
    valid = col < 7
    zm = jnp.where(valid, z, -jnp.inf)
    m = jnp.max(zm, axis=1, keepdims=True)
    ssum = jnp.sum(jnp.where(valid, jnp.exp(z - m), 0.0),
                   axis=1, keepdims=True)
    o_ref[...] = z - m - jnp.log(ssum)


def head(h, lw1, lb1, lw2, lb2):
    dmid = lw1.shape[1]
    lw1c = lw1.reshape(8, 128, dmid).astype(jnp.bfloat16)
    lw2p = (jnp.zeros((dmid, 128), jnp.float32).at[:, :7].set(lw2)
            .astype(jnp.bfloat16))
    lb2p = jnp.zeros((1, 128), jnp.float32).at[0, :7].set(lb2)
    out = pl.pallas_call(
        _head_body,
        grid=(NBLK,),
        in_specs=[
            pl.BlockSpec((8, BLK, 128), lambda i: (0, i, 0)),
            pl.BlockSpec((8, 128, dmid), lambda i: (0, 0, 0)),
            pl.BlockSpec((1, dmid), lambda i: (0, 0)),
            pl.BlockSpec((dmid, 128), lambda i: (0, 0)),
            pl.BlockSpec((1, 128), lambda i: (0, 0)),
        ],
        out_specs=pl.BlockSpec((BLK, 128), lambda i: (i, 0)),
        out_shape=jax.ShapeDtypeStruct((NPAD, 128), jnp.float32),
    )(h, lw1c, lb1.reshape(1, -1), lw2p, lb2p)
    return out[:NNODES, :7]


def _weight_chunks(W, a_src, a_dst):
    d = W.shape[0]
    aux = jnp.zeros((d, 128), jnp.float32)
    aux = aux.at[:, 0].set(W @ a_src).at[:, 1].set(W @ a_dst)
    w_aug = jnp.concatenate([W, aux], axis=1)
    return (w_aug.reshape(d // 128, 128, 9, 128).transpose(0, 2, 1, 3)
            .astype(jnp.bfloat16))


def gat_layer(xc, edges, zf, zf16, W, a_src, a_dst, b):
    wc = _weight_chunks(W, a_src, a_dst)
    hc = matmul_chunked(xc, wc)
    av = jnp.stack([hc[8, :, 0], hc[8, :, 1]])
    hcflat = hc.reshape(9 * NPAD * 2, CW)
    osc, dpart = _gat_sc(hcflat, av, edges, zf, zf16)
    return combine_bias_stats(osc, dpart, b.reshape(8, 1, 128))


def _bn_stats(s, q):
    mean = s / NNODES
    var = q / NNODES - mean * mean
    return mean, jax.lax.rsqrt(var + 1e-5)


def kernel(x, edge_index, W1, a_src1, a_dst1, b1, g1, beta1,
           W2, a_src2, a_dst2, b2, g2, beta2, lw1, lb1, lw2, lb2):
    loop = jnp.arange(NNODES, dtype=edge_index.dtype)
    src = jnp.concatenate([edge_index[0], loop])
    dst = jnp.concatenate([edge_index[1], loop])
    pad = ETOT - src.shape[0]
    src = jnp.pad(src, (0, pad))
    dst = jnp.pad(dst, (0, pad))
    edges = jnp.stack([src, dst]).reshape(2, NW, NB, EB)
    zf = jnp.zeros((NSLICE, CW), jnp.float32)
    zf16 = jnp.zeros((NSLICE, 16), jnp.float32)

    xc = jnp.pad(x, ((0, NPAD - NNODES), (0, 0)))[None].astype(jnp.bfloat16)
    h, s, q = gat_layer(xc, edges, zf, zf16, W1, a_src1, a_dst1, b1)
    mean, rstd = _bn_stats(s, q)
    h = bn_apply(h, mean, rstd, g1.reshape(8, 1, 128), beta1.reshape(8, 1, 128))
    h, s, q = gat_layer(h, edges, zf, zf16, W2, a_src2, a_dst2, b2)
    mean, rstd = _bn_stats(s, q)
    h = bn_apply(h, mean, rstd, g2.reshape(8, 1, 128), beta2.reshape(8, 1, 128))
    return head(h, lw1, lb1, lw2, lb2)
